# baseline (speedup 1.0000x reference)
import functools

import jax
import jax.numpy as jnp
import numpy as np
from jax import lax
from jax.experimental import pallas as pl
from jax.experimental.pallas import tpu as pltpu
from jax.experimental.pallas import tpu_sc as plsc

NUM_TIMESTEPS = 1000
BETA_START = 1e-4
BETA_END = 0.02

ROWS_PER_BLOCK = 42
_TPAD = 1024
_LANES = 16
_DROW = 128
_NWORKERS = 16


def _f32_cumprod_tables():
    betas = np.linspace(BETA_START, BETA_END, NUM_TIMESTEPS, dtype=np.float32)
    ac = np.cumprod((1.0 - betas).astype(np.float32), dtype=np.float32)
    sqrt_ac = np.sqrt(ac).astype(np.float32)
    sqrt_1mac = np.sqrt((1.0 - ac).astype(np.float32)).astype(np.float32)
    tabs = np.zeros((_TPAD, _DROW), dtype=np.float32)
    tabs[:NUM_TIMESTEPS, 0] = sqrt_ac
    tabs[:NUM_TIMESTEPS, 1] = sqrt_1mac
    return tabs


def _sc_lookup(tab_hbm, ts_hbm, out_hbm, ts_v, rows_v, sem):
    nc = plsc.get_sparse_core_info().num_cores
    wid = lax.axis_index("s") * nc + lax.axis_index("c")

    @pl.when(wid < _NWORKERS)
    def _():
        base = wid * _LANES
        pltpu.sync_copy(ts_hbm.at[pl.ds(base, _LANES)], ts_v)
        pltpu.async_copy(tab_hbm.at[ts_v], rows_v, sem).wait()
        pltpu.sync_copy(rows_v, out_hbm.at[pl.ds(base, _LANES), :])


def _tc_body(scale_ref, x_ref, n_ref, o_ref, tr_ref):
    @pl.when(pl.program_id(0) == 0)
    def _():
        tr_ref[...] = jnp.swapaxes(scale_ref[...], 0, 1)

    a = tr_ref[0:1, :][None]
    c = tr_ref[1:2, :][None]
    o_ref[...] = a * x_ref[...] + c * n_ref[...]


def kernel(original_samples, noise, timesteps):
    B, C, H, W = original_samples.shape
    R = C * H
    Rb = ROWS_PER_BLOCK
    x = jnp.transpose(original_samples, (1, 2, 3, 0)).reshape(R, W, B)
    n = jnp.transpose(noise, (1, 2, 3, 0)).reshape(R, W, B)
    tabs = jnp.asarray(_f32_cumprod_tables())
    ts = timesteps.astype(jnp.int32)

    sc_lookup = functools.partial(
        pl.kernel,
        mesh=plsc.VectorSubcoreMesh(core_axis_name="c", subcore_axis_name="s"),
        out_type=jax.ShapeDtypeStruct((B, _DROW), jnp.float32),
        scratch_types=[
            pltpu.VMEM((_LANES,), jnp.int32),
            pltpu.VMEM((_LANES, _DROW), jnp.float32),
            pltpu.SemaphoreType.DMA,
        ],
    )(_sc_lookup)
    scales = sc_lookup(tabs, ts)

    out = pl.pallas_call(
        _tc_body,
        grid=(R // Rb,),
        in_specs=[
            pl.BlockSpec((B, _DROW), lambda i: (0, 0)),
            pl.BlockSpec((Rb, W, B), lambda i: (i, 0, 0)),
            pl.BlockSpec((Rb, W, B), lambda i: (i, 0, 0)),
        ],
        out_specs=pl.BlockSpec((Rb, W, B), lambda i: (i, 0, 0)),
        out_shape=jax.ShapeDtypeStruct((R, W, B), jnp.float32),
        scratch_shapes=[pltpu.VMEM((_DROW, B), jnp.float32)],
        compiler_params=pltpu.CompilerParams(
            dimension_semantics=("arbitrary",),
        ),
    )(scales, x, n)
    return jnp.transpose(out.reshape(C, H, W, B), (3, 0, 1, 2))


# speedup vs baseline: 1.1599x; 1.1599x over previous
import jax
import jax.numpy as jnp
import numpy as np
from jax import lax
from jax.experimental import pallas as pl
from jax.experimental.pallas import tpu as pltpu

NUM_TIMESTEPS = 1000
BETA_START = 1e-4
BETA_END = 0.02

ROWS_PER_BLOCK = 42
_TPAD = 1024


def _f32_cumprod_tables():
    betas = np.linspace(BETA_START, BETA_END, NUM_TIMESTEPS, dtype=np.float32)
    ac = np.cumprod((1.0 - betas).astype(np.float32), dtype=np.float32)
    sqrt_ac = np.sqrt(ac).astype(np.float32)
    sqrt_1mac = np.sqrt((1.0 - ac).astype(np.float32)).astype(np.float32)
    tabs = np.zeros((2, _TPAD), dtype=np.float32)
    tabs[0, :NUM_TIMESTEPS] = sqrt_ac
    tabs[1, :NUM_TIMESTEPS] = sqrt_1mac
    return tabs


def _body(ts_ref, tab_ref, x_ref, n_ref, o_ref, scale_ref):
    @pl.when(pl.program_id(0) == 0)
    def _gather_scales():
        ts = ts_ref[...]
        iota = lax.broadcasted_iota(jnp.int32, (_TPAD, ts.shape[1]), 0)
        onehot = (iota == ts).astype(jnp.float32)
        scale_ref[...] = jnp.dot(
            tab_ref[...], onehot,
            preferred_element_type=jnp.float32,
            precision=lax.Precision.HIGHEST,
        )

    a = scale_ref[0:1, :][None]
    c = scale_ref[1:2, :][None]
    o_ref[...] = a * x_ref[...] + c * n_ref[...]


def kernel(original_samples, noise, timesteps):
    B, C, H, W = original_samples.shape
    R = C * H
    Rb = ROWS_PER_BLOCK
    x = jnp.transpose(original_samples, (1, 2, 3, 0)).reshape(R, W, B)
    n = jnp.transpose(noise, (1, 2, 3, 0)).reshape(R, W, B)
    tabs = jnp.asarray(_f32_cumprod_tables())
    ts = timesteps.astype(jnp.int32).reshape(1, B)

    out = pl.pallas_call(
        _body,
        grid=(R // Rb,),
        in_specs=[
            pl.BlockSpec((1, B), lambda i: (0, 0)),
            pl.BlockSpec((2, _TPAD), lambda i: (0, 0)),
            pl.BlockSpec((Rb, W, B), lambda i: (i, 0, 0)),
            pl.BlockSpec((Rb, W, B), lambda i: (i, 0, 0)),
        ],
        out_specs=pl.BlockSpec((Rb, W, B), lambda i: (i, 0, 0)),
        out_shape=jax.ShapeDtypeStruct((R, W, B), jnp.float32),
        scratch_shapes=[pltpu.VMEM((2, B), jnp.float32)],
        compiler_params=pltpu.CompilerParams(
            dimension_semantics=("arbitrary",),
        ),
    )(ts, tabs, x, n)
    return jnp.transpose(out.reshape(C, H, W, B), (3, 0, 1, 2))
